# baseline (speedup 1.0000x reference)
import functools

import jax
import jax.numpy as jnp
from jax import lax
from jax.experimental import pallas as pl
from jax.experimental.pallas import tpu as pltpu
from jax.experimental.pallas import tpu_sc as plsc

VOCAB = 1000000
EMBED_DIM = 64
BATCH = 4096
SEQ = 200
S_A = 104
S_B = SEQ - S_A
NFULL = SEQ // 16
TAIL_OFF = SEQ - 16

_info = plsc.get_sparse_core_info()
NC = _info.num_cores
NS = _info.num_subcores
NW = NC * NS
DPW = BATCH // NW

G = 4
NG = DPW // G

_mesh = plsc.VectorSubcoreMesh(core_axis_name="c", subcore_axis_name="s")


def _matvec_body(xt_ref, w_ref, o_ref):
    o_ref[...] = jnp.sum(xt_ref[...] * w_ref[...], axis=0)


_ROWS_PER_BLOCK = 16384


def _table_matvec(table_t, w_col):
    return pl.pallas_call(
        _matvec_body,
        grid=(pl.cdiv(VOCAB, _ROWS_PER_BLOCK),),
        in_specs=[
            pl.BlockSpec((EMBED_DIM, _ROWS_PER_BLOCK), lambda i: (0, i)),
            pl.BlockSpec((EMBED_DIM, 1), lambda i: (0, 0)),
        ],
        out_specs=pl.BlockSpec((_ROWS_PER_BLOCK,), lambda i: (i,)),
        out_shape=jax.ShapeDtypeStruct((VOCAB,), jnp.float32),
    )(table_t, w_col)


def _doc_reduce(ref, d, zero_is_pad):
    acc = jnp.zeros((16,), jnp.float32)
    for k in range(NFULL):
        v = ref[d, pl.ds(k * 16, 16)]
        if zero_is_pad:
            acc = acc + jnp.where(v != 0, 1.0, 0.0).astype(jnp.float32)
        else:
            acc = acc + v
    lane = lax.iota(jnp.int32, 16)
    tail = ref[d, pl.ds(TAIL_OFF, 16)]
    if zero_is_pad:
        t = jnp.where((lane >= 8) & (tail != 0), 1.0, 0.0).astype(jnp.float32)
    else:
        t = jnp.where(lane >= 8, tail, jnp.zeros((16,), jnp.float32))
    return jnp.sum(acc + t)


@functools.partial(
    pl.kernel,
    mesh=_mesh,
    out_type=jax.ShapeDtypeStruct((BATCH,), jnp.float32),
    scratch_types=[
        pltpu.VMEM((DPW, SEQ), jnp.int32),
        pltpu.VMEM((DPW, SEQ), jnp.float32),
        pltpu.VMEM((DPW,), jnp.float32),
        pltpu.VMEM((16,), jnp.float32),
        pltpu.VMEM((16,), jnp.float32),
        pltpu.VMEM_SHARED((VOCAB,), jnp.float32),
        pltpu.SemaphoreType.DMA,
        pltpu.SemaphoreType.DMA,
    ],
    compiler_params=pltpu.CompilerParams(needs_layout_passes=False,
                                         use_tc_tiling_on_sc=False),
)
def _pool_kernel(docs_hbm, tw_hbm, b_hbm, out_hbm,
                 idx_v, vals_v, out_v, tw0_v, b_v, tw_sh, sem0, sem1):
    cid = lax.axis_index("c")
    sid = lax.axis_index("s")
    wid = sid * NC + cid
    base = wid * DPW
    sems = (sem0, sem1)

    pltpu.sync_copy(docs_hbm.at[pl.ds(base, DPW)], idx_v)
    pltpu.sync_copy(tw_hbm.at[pl.ds(0, 16)], tw0_v)
    pltpu.sync_copy(b_hbm, b_v)

    @pl.when(sid == 0)
    def _():
        pltpu.sync_copy(tw_hbm, tw_sh)
    plsc.subcore_barrier()

    bvec = b_v[pl.ds(0, 16)]
    tw0 = jnp.full((16,), tw0_v[pl.ds(0, 16)][0], jnp.float32)
    lane = lax.iota(jnp.int32, 16)
    onev = jnp.full((16,), 1.0, jnp.float32)
    seqv = jnp.full((16,), jnp.float32(SEQ), jnp.float32)

    def _group_streams(g, p):
        for j in range(G):
            d = g * G + j
            for off, ln in ((0, S_A), (S_A, S_B)):
                yield pltpu.make_async_copy(
                    tw_sh.at[idx_v.at[d, pl.ds(off, ln)]],
                    vals_v.at[d, pl.ds(off, ln)],
                    sems[p])

    def issue(g, p):
        for cp in _group_streams(g, p):
            cp.start()

    def drain(g, p):
        for cp in _group_streams(g, p):
            cp.wait()

    def process(g):
        for j in range(G):
            d = g * G + j
            s = _doc_reduce(vals_v, d, zero_is_pad=False)
            count = _doc_reduce(idx_v, d, zero_is_pad=True)
            countv = jnp.full((16,), count, jnp.float32)
            n0v = seqv - countv
            invv = 1.0 / jnp.maximum(countv, onev)
            lv = (jnp.full((16,), s, jnp.float32) - n0v * tw0) * invv + bvec
            e = jnp.exp(lv - lv)
            val = e / e
            plsc.store_scatter(out_v, [jnp.full((16,), d, jnp.int32)], val,
                               mask=lane == 0)

    issue(0, 0)
    issue(1, 1)

    def pair_body(i, carry):
        g0 = i * 2
        for p in range(2):
            g = g0 + p
            drain(g, p)

            @pl.when(g + 2 < NG)
            def _():
                issue(g + 2, p)

            process(g)
        return carry

    lax.fori_loop(0, NG // 2, pair_body, 0)
    pltpu.sync_copy(out_v, out_hbm.at[pl.ds(base, DPW)])


def kernel(documents, table, W, b):
    tw = _table_matvec(table.T, W)
    out = _pool_kernel(documents.astype(jnp.int32), tw,
                       jnp.full((16,), b[0], jnp.float32))
    return out.reshape(BATCH, 1)


# speedup vs baseline: 1.1218x; 1.1218x over previous
import functools

import jax
import jax.numpy as jnp
from jax import lax
from jax.experimental import pallas as pl
from jax.experimental.pallas import tpu as pltpu
from jax.experimental.pallas import tpu_sc as plsc

VOCAB = 1000000
EMBED_DIM = 64
BATCH = 4096
SEQ = 200
S_A = 104
S_B = SEQ - S_A
NFULL = SEQ // 16
TAIL_OFF = SEQ - 16

_info = plsc.get_sparse_core_info()
NC = _info.num_cores
NS = _info.num_subcores
NW = NC * NS
DPW = BATCH // NW

G = 4
NG = DPW // G

_mesh = plsc.VectorSubcoreMesh(core_axis_name="c", subcore_axis_name="s")


def _matvec_body(xt_ref, w_ref, o_ref):
    o_ref[...] = jnp.sum(xt_ref[...] * w_ref[...], axis=0)


_ROWS_PER_BLOCK = 40960


def _table_matvec(table_t, w_col):
    return pl.pallas_call(
        _matvec_body,
        grid=(pl.cdiv(VOCAB, _ROWS_PER_BLOCK),),
        in_specs=[
            pl.BlockSpec((EMBED_DIM, _ROWS_PER_BLOCK), lambda i: (0, i)),
            pl.BlockSpec((EMBED_DIM, 1), lambda i: (0, 0)),
        ],
        out_specs=pl.BlockSpec((_ROWS_PER_BLOCK,), lambda i: (i,)),
        out_shape=jax.ShapeDtypeStruct((VOCAB,), jnp.float32),
    )(table_t, w_col)


def _doc_reduce(ref, d, zero_is_pad):
    acc = jnp.zeros((16,), jnp.float32)
    for k in range(NFULL):
        v = ref[d, pl.ds(k * 16, 16)]
        if zero_is_pad:
            acc = acc + jnp.where(v != 0, 1.0, 0.0).astype(jnp.float32)
        else:
            acc = acc + v
    lane = lax.iota(jnp.int32, 16)
    tail = ref[d, pl.ds(TAIL_OFF, 16)]
    if zero_is_pad:
        t = jnp.where((lane >= 8) & (tail != 0), 1.0, 0.0).astype(jnp.float32)
    else:
        t = jnp.where(lane >= 8, tail, jnp.zeros((16,), jnp.float32))
    return jnp.sum(acc + t)


@functools.partial(
    pl.kernel,
    mesh=_mesh,
    out_type=jax.ShapeDtypeStruct((BATCH,), jnp.float32),
    scratch_types=[
        pltpu.VMEM((DPW, SEQ), jnp.int32),
        pltpu.VMEM((DPW, SEQ), jnp.float32),
        pltpu.VMEM((DPW,), jnp.float32),
        pltpu.VMEM((16,), jnp.float32),
        pltpu.VMEM((16,), jnp.float32),
        pltpu.VMEM_SHARED((VOCAB,), jnp.float32),
        pltpu.SemaphoreType.DMA,
        pltpu.SemaphoreType.DMA,
    ],
    compiler_params=pltpu.CompilerParams(needs_layout_passes=False,
                                         use_tc_tiling_on_sc=False),
)
def _pool_kernel(docs_hbm, tw_hbm, b_hbm, out_hbm,
                 idx_v, vals_v, out_v, tw0_v, b_v, tw_sh, sem0, sem1):
    cid = lax.axis_index("c")
    sid = lax.axis_index("s")
    wid = sid * NC + cid
    base = wid * DPW
    sems = (sem0, sem1)

    pltpu.sync_copy(docs_hbm.at[pl.ds(base, DPW)], idx_v)
    pltpu.sync_copy(tw_hbm.at[pl.ds(0, 16)], tw0_v)
    pltpu.sync_copy(b_hbm, b_v)

    @pl.when(sid == 0)
    def _():
        pltpu.sync_copy(tw_hbm, tw_sh)
    plsc.subcore_barrier()

    bvec = b_v[pl.ds(0, 16)]
    tw0 = jnp.full((16,), tw0_v[pl.ds(0, 16)][0], jnp.float32)
    lane = lax.iota(jnp.int32, 16)
    onev = jnp.full((16,), 1.0, jnp.float32)
    seqv = jnp.full((16,), jnp.float32(SEQ), jnp.float32)

    def _group_streams(g, p):
        for j in range(G):
            d = g * G + j
            for off, ln in ((0, S_A), (S_A, S_B)):
                yield pltpu.make_async_copy(
                    tw_sh.at[idx_v.at[d, pl.ds(off, ln)]],
                    vals_v.at[d, pl.ds(off, ln)],
                    sems[p])

    def issue(g, p):
        for cp in _group_streams(g, p):
            cp.start()

    def drain(g, p):
        for cp in _group_streams(g, p):
            cp.wait()

    def process(g):
        for j in range(G):
            d = g * G + j
            s = _doc_reduce(vals_v, d, zero_is_pad=False)
            count = _doc_reduce(idx_v, d, zero_is_pad=True)
            countv = jnp.full((16,), count, jnp.float32)
            n0v = seqv - countv
            invv = 1.0 / jnp.maximum(countv, onev)
            lv = (jnp.full((16,), s, jnp.float32) - n0v * tw0) * invv + bvec
            e = jnp.exp(lv - lv)
            val = e / e
            plsc.store_scatter(out_v, [jnp.full((16,), d, jnp.int32)], val,
                               mask=lane == 0)

    issue(0, 0)
    issue(1, 1)

    def pair_body(i, carry):
        g0 = i * 2
        for p in range(2):
            g = g0 + p
            drain(g, p)

            @pl.when(g + 2 < NG)
            def _():
                issue(g + 2, p)

            process(g)
        return carry

    lax.fori_loop(0, NG // 2, pair_body, 0)
    pltpu.sync_copy(out_v, out_hbm.at[pl.ds(base, DPW)])


def kernel(documents, table, W, b):
    tw = _table_matvec(table.T, W)
    out = _pool_kernel(documents.astype(jnp.int32), tw,
                       jnp.full((16,), b[0], jnp.float32))
    return out.reshape(BATCH, 1)


# speedup vs baseline: 1.1262x; 1.0039x over previous
import functools

import jax
import jax.numpy as jnp
from jax import lax
from jax.experimental import pallas as pl
from jax.experimental.pallas import tpu as pltpu
from jax.experimental.pallas import tpu_sc as plsc

VOCAB = 1000000
EMBED_DIM = 64
BATCH = 4096
SEQ = 200
S_A = 104
S_B = SEQ - S_A
NFULL = SEQ // 16
TAIL_OFF = SEQ - 16

_info = plsc.get_sparse_core_info()
NC = _info.num_cores
NS = _info.num_subcores
NW = NC * NS
DPW = BATCH // NW

G = 8
NG = DPW // G
STAGERS = 8
STAGE_CHUNK = VOCAB // STAGERS

_mesh = plsc.VectorSubcoreMesh(core_axis_name="c", subcore_axis_name="s")


def _matvec_body(xt_ref, w_ref, o_ref):
    o_ref[...] = jnp.sum(xt_ref[...] * w_ref[...], axis=0)


_ROWS_PER_BLOCK = 32768


def _table_matvec(table_t, w_col):
    return pl.pallas_call(
        _matvec_body,
        grid=(pl.cdiv(VOCAB, _ROWS_PER_BLOCK),),
        in_specs=[
            pl.BlockSpec((EMBED_DIM, _ROWS_PER_BLOCK), lambda i: (0, i)),
            pl.BlockSpec((EMBED_DIM, 1), lambda i: (0, 0)),
        ],
        out_specs=pl.BlockSpec((_ROWS_PER_BLOCK,), lambda i: (i,)),
        out_shape=jax.ShapeDtypeStruct((VOCAB,), jnp.float32),
    )(table_t, w_col)


def _doc_reduce(ref, d, zero_is_pad):
    acc = jnp.zeros((16,), jnp.float32)
    for k in range(NFULL):
        v = ref[d, pl.ds(k * 16, 16)]
        if zero_is_pad:
            acc = acc + jnp.where(v != 0, 1.0, 0.0).astype(jnp.float32)
        else:
            acc = acc + v
    lane = lax.iota(jnp.int32, 16)
    tail = ref[d, pl.ds(TAIL_OFF, 16)]
    if zero_is_pad:
        t = jnp.where((lane >= 8) & (tail != 0), 1.0, 0.0).astype(jnp.float32)
    else:
        t = jnp.where(lane >= 8, tail, jnp.zeros((16,), jnp.float32))
    return jnp.sum(acc + t)


@functools.partial(
    pl.kernel,
    mesh=_mesh,
    out_type=jax.ShapeDtypeStruct((BATCH,), jnp.float32),
    scratch_types=[
        pltpu.VMEM((DPW, SEQ), jnp.int32),
        pltpu.VMEM((DPW, SEQ), jnp.float32),
        pltpu.VMEM((DPW,), jnp.float32),
        pltpu.VMEM((16,), jnp.float32),
        pltpu.VMEM((16,), jnp.float32),
        pltpu.VMEM_SHARED((VOCAB,), jnp.float32),
        pltpu.SemaphoreType.DMA,
        pltpu.SemaphoreType.DMA,
        pltpu.SemaphoreType.DMA,
    ],
    compiler_params=pltpu.CompilerParams(needs_layout_passes=False,
                                         use_tc_tiling_on_sc=False),
)
def _pool_kernel(docs_hbm, tw_hbm, b_hbm, out_hbm,
                 idx_v, vals_v, out_v, tw0_v, b_v, tw_sh, sem0, sem1, sem2):
    cid = lax.axis_index("c")
    sid = lax.axis_index("s")
    wid = sid * NC + cid
    base = wid * DPW
    sems = (sem0, sem1)

    soff = jnp.minimum(sid, STAGERS - 1) * STAGE_CHUNK
    stage_cp = pltpu.make_async_copy(
        tw_hbm.at[pl.ds(soff, STAGE_CHUNK)],
        tw_sh.at[pl.ds(soff, STAGE_CHUNK)], sem2)

    @pl.when(sid < STAGERS)
    def _():
        stage_cp.start()

    pltpu.sync_copy(docs_hbm.at[pl.ds(base, DPW)], idx_v)
    pltpu.sync_copy(tw_hbm.at[pl.ds(0, 16)], tw0_v)
    pltpu.sync_copy(b_hbm, b_v)

    @pl.when(sid < STAGERS)
    def _():
        stage_cp.wait()
    plsc.subcore_barrier()

    bvec = b_v[pl.ds(0, 16)]
    tw0 = jnp.full((16,), tw0_v[pl.ds(0, 16)][0], jnp.float32)
    lane = lax.iota(jnp.int32, 16)
    onev = jnp.full((16,), 1.0, jnp.float32)
    seqv = jnp.full((16,), jnp.float32(SEQ), jnp.float32)

    def _group_streams(g, p):
        for j in range(G):
            d = g * G + j
            for off, ln in ((0, S_A), (S_A, S_B)):
                yield pltpu.make_async_copy(
                    tw_sh.at[idx_v.at[d, pl.ds(off, ln)]],
                    vals_v.at[d, pl.ds(off, ln)],
                    sems[p])

    def issue(g, p):
        for cp in _group_streams(g, p):
            cp.start()

    def drain(g, p):
        for cp in _group_streams(g, p):
            cp.wait()

    def process(g):
        for j in range(G):
            d = g * G + j
            s = _doc_reduce(vals_v, d, zero_is_pad=False)
            count = _doc_reduce(idx_v, d, zero_is_pad=True)
            countv = jnp.full((16,), count, jnp.float32)
            n0v = seqv - countv
            invv = 1.0 / jnp.maximum(countv, onev)
            lv = (jnp.full((16,), s, jnp.float32) - n0v * tw0) * invv + bvec
            e = jnp.exp(lv - lv)
            val = e / e
            plsc.store_scatter(out_v, [jnp.full((16,), d, jnp.int32)], val,
                               mask=lane == 0)

    issue(0, 0)
    issue(1, 1)

    def pair_body(i, carry):
        g0 = i * 2
        for p in range(2):
            g = g0 + p
            drain(g, p)

            @pl.when(g + 2 < NG)
            def _():
                issue(g + 2, p)

            process(g)
        return carry

    lax.fori_loop(0, NG // 2, pair_body, 0)
    pltpu.sync_copy(out_v, out_hbm.at[pl.ds(base, DPW)])


def kernel(documents, table, W, b):
    tw = _table_matvec(table.T, W)
    out = _pool_kernel(documents.astype(jnp.int32), tw,
                       jnp.full((16,), b[0], jnp.float32))
    return out.reshape(BATCH, 1)


# speedup vs baseline: 1.1738x; 1.0423x over previous
import functools

import jax
import jax.numpy as jnp
from jax import lax
from jax.experimental import pallas as pl
from jax.experimental.pallas import tpu as pltpu
from jax.experimental.pallas import tpu_sc as plsc

VOCAB = 1000000
EMBED_DIM = 64
BATCH = 4096
SEQ = 200
S_A = 104
S_B = SEQ - S_A
NFULL = SEQ // 16
TAIL_OFF = SEQ - 16

_info = plsc.get_sparse_core_info()
NC = _info.num_cores
NS = _info.num_subcores
NW = NC * NS
DPW = BATCH // NW

GT = 10
NG = SEQ // GT
STAGERS = 8
STAGE_CHUNK = VOCAB // STAGERS

_mesh = plsc.VectorSubcoreMesh(core_axis_name="c", subcore_axis_name="s")


def _matvec_body(xt_ref, w_ref, o_ref):
    o_ref[...] = jnp.sum(xt_ref[...] * w_ref[...], axis=0)


_ROWS_PER_BLOCK = 32768


def _table_matvec(table_t, w_col):
    return pl.pallas_call(
        _matvec_body,
        grid=(pl.cdiv(VOCAB, _ROWS_PER_BLOCK),),
        in_specs=[
            pl.BlockSpec((EMBED_DIM, _ROWS_PER_BLOCK), lambda i: (0, i)),
            pl.BlockSpec((EMBED_DIM, 1), lambda i: (0, 0)),
        ],
        out_specs=pl.BlockSpec((_ROWS_PER_BLOCK,), lambda i: (i,)),
        out_shape=jax.ShapeDtypeStruct((VOCAB,), jnp.float32),
    )(table_t, w_col)


@functools.partial(
    pl.kernel,
    mesh=_mesh,
    out_type=jax.ShapeDtypeStruct((BATCH,), jnp.float32),
    scratch_types=[
        pltpu.VMEM((SEQ, DPW), jnp.int32),
        pltpu.VMEM((SEQ, DPW), jnp.float32),
        pltpu.VMEM((DPW,), jnp.float32),
        pltpu.VMEM((16,), jnp.float32),
        pltpu.VMEM((16,), jnp.float32),
        pltpu.VMEM_SHARED((VOCAB,), jnp.float32),
        pltpu.SemaphoreType.DMA,
        pltpu.SemaphoreType.DMA,
        pltpu.SemaphoreType.DMA,
    ],
    compiler_params=pltpu.CompilerParams(needs_layout_passes=False,
                                         use_tc_tiling_on_sc=False),
)
def _pool_kernel(docs_t_hbm, tw_hbm, b_hbm, out_hbm,
                 idx_v, vals_v, out_v, tw0_v, b_v, tw_sh, sem0, sem1, sem2):
    cid = lax.axis_index("c")
    sid = lax.axis_index("s")
    wid = sid * NC + cid
    base = wid * DPW
    sems = (sem0, sem1)

    soff = jnp.minimum(sid, STAGERS - 1) * STAGE_CHUNK
    stage_cp = pltpu.make_async_copy(
        tw_hbm.at[pl.ds(soff, STAGE_CHUNK)],
        tw_sh.at[pl.ds(soff, STAGE_CHUNK)], sem2)

    @pl.when(sid < STAGERS)
    def _():
        stage_cp.start()

    pltpu.sync_copy(docs_t_hbm.at[:, pl.ds(base, DPW)], idx_v)
    pltpu.sync_copy(tw_hbm.at[pl.ds(0, 16)], tw0_v)
    pltpu.sync_copy(b_hbm, b_v)

    @pl.when(sid < STAGERS)
    def _():
        stage_cp.wait()
    plsc.subcore_barrier()

    bvec = b_v[pl.ds(0, 16)]
    tw0 = jnp.full((16,), tw0_v[pl.ds(0, 16)][0], jnp.float32)
    onev = jnp.full((16,), 1.0, jnp.float32)
    seqv = jnp.full((16,), jnp.float32(SEQ), jnp.float32)
    zerov = jnp.zeros((16,), jnp.float32)
    NL = DPW // 16

    def _group_streams(g, p):
        for j in range(GT):
            t = g * GT + j
            yield pltpu.make_async_copy(
                tw_sh.at[idx_v.at[t]], vals_v.at[t], sems[p])

    def issue(g, p):
        for cp in _group_streams(g, p):
            cp.start()

    def drain(g, p):
        for cp in _group_streams(g, p):
            cp.wait()

    def process(g, carry):
        acc, cnt = carry
        for j in range(GT):
            t = g * GT + j
            acc = tuple(acc[m] + vals_v[t, pl.ds(m * 16, 16)]
                        for m in range(NL))
            cnt = tuple(
                cnt[m]
                + jnp.where(idx_v[t, pl.ds(m * 16, 16)] != 0, 1.0, 0.0)
                .astype(jnp.float32)
                for m in range(NL))
        return acc, cnt

    issue(0, 0)
    issue(1, 1)

    def pair_body(i, carry):
        g0 = i * 2
        for p in range(2):
            g = g0 + p
            drain(g, p)

            @pl.when(g + 2 < NG)
            def _():
                issue(g + 2, p)

            carry = process(g, carry)
        return carry

    zeros = tuple(zerov for _ in range(NL))
    acc, cnt = lax.fori_loop(0, NG // 2, pair_body, (zeros, zeros))

    for m in range(NL):
        countv = cnt[m]
        n0v = seqv - countv
        invv = 1.0 / jnp.maximum(countv, onev)
        lv = (acc[m] - n0v * tw0) * invv + bvec
        e = jnp.exp(lv - lv)
        out_v[pl.ds(m * 16, 16)] = e / e

    pltpu.sync_copy(out_v, out_hbm.at[pl.ds(base, DPW)])


def kernel(documents, table, W, b):
    tw = _table_matvec(table.T, W)
    out = _pool_kernel(documents.astype(jnp.int32).T, tw,
                       jnp.full((16,), b[0], jnp.float32))
    return out.reshape(BATCH, 1)


# speedup vs baseline: 1.1881x; 1.0122x over previous
import functools

import jax
import jax.numpy as jnp
from jax import lax
from jax.experimental import pallas as pl
from jax.experimental.pallas import tpu as pltpu
from jax.experimental.pallas import tpu_sc as plsc

VOCAB = 1000000
EMBED_DIM = 64
BATCH = 4096
SEQ = 200

_info = plsc.get_sparse_core_info()
NC = _info.num_cores
NS = _info.num_subcores
NW = NC * NS
DPW = BATCH // NW

GT = 10
NG = SEQ // GT
STAGERS = 8
STAGE_CHUNK = VOCAB // STAGERS

_mesh = plsc.VectorSubcoreMesh(core_axis_name="c", subcore_axis_name="s")


def _matvec_body(xt_ref, w_ref, o_ref):
    o_ref[...] = jnp.sum(xt_ref[...] * w_ref[...], axis=0)


_ROWS_PER_BLOCK = 32768


def _table_matvec(table_t, w_col):
    return pl.pallas_call(
        _matvec_body,
        grid=(pl.cdiv(VOCAB, _ROWS_PER_BLOCK),),
        in_specs=[
            pl.BlockSpec((EMBED_DIM, _ROWS_PER_BLOCK), lambda i: (0, i)),
            pl.BlockSpec((EMBED_DIM, 1), lambda i: (0, 0)),
        ],
        out_specs=pl.BlockSpec((_ROWS_PER_BLOCK,), lambda i: (i,)),
        out_shape=jax.ShapeDtypeStruct((VOCAB,), jnp.float32),
    )(table_t, w_col)


@functools.partial(
    pl.kernel,
    mesh=_mesh,
    out_type=jax.ShapeDtypeStruct((BATCH,), jnp.float32),
    scratch_types=[
        pltpu.VMEM((SEQ, DPW), jnp.int32),
        pltpu.VMEM((SEQ, DPW), jnp.float32),
        pltpu.VMEM((DPW,), jnp.float32),
        pltpu.VMEM((16,), jnp.float32),
        pltpu.VMEM((16,), jnp.float32),
        pltpu.VMEM_SHARED((VOCAB,), jnp.float32),
        pltpu.SemaphoreType.DMA,
        pltpu.SemaphoreType.DMA,
        pltpu.SemaphoreType.DMA,
    ],
    compiler_params=pltpu.CompilerParams(needs_layout_passes=False,
                                         use_tc_tiling_on_sc=False),
)
def _pool_kernel(docs_t_hbm, tw_hbm, b_hbm, out_hbm,
                 idx_v, vals_v, out_v, tw0_v, b_v, tw_sh, sem0, sem1, sem2):
    cid = lax.axis_index("c")
    sid = lax.axis_index("s")
    wid = sid * NC + cid
    base = wid * DPW
    sems = (sem0, sem1)

    soff = jnp.minimum(sid, STAGERS - 1) * STAGE_CHUNK
    stage_cp = pltpu.make_async_copy(
        tw_hbm.at[pl.ds(soff, STAGE_CHUNK)],
        tw_sh.at[pl.ds(soff, STAGE_CHUNK)], sem2)

    @pl.when(sid < STAGERS)
    def _():
        stage_cp.start()

    pltpu.sync_copy(docs_t_hbm.at[:, pl.ds(base, DPW)], idx_v)
    pltpu.sync_copy(tw_hbm.at[pl.ds(0, 16)], tw0_v)
    pltpu.sync_copy(b_hbm, b_v)

    @pl.when(sid < STAGERS)
    def _():
        stage_cp.wait()
    plsc.subcore_barrier()

    bvec = b_v[pl.ds(0, 16)]
    tw0 = jnp.full((16,), tw0_v[pl.ds(0, 16)][0], jnp.float32)
    onev = jnp.full((16,), 1.0, jnp.float32)
    seqv = jnp.full((16,), jnp.float32(SEQ), jnp.float32)
    zerov = jnp.zeros((16,), jnp.float32)
    NL = DPW // 16

    def _group_streams(g, p):
        for j in range(GT):
            t = g * GT + j
            yield pltpu.make_async_copy(
                tw_sh.at[idx_v.at[t]], vals_v.at[t], sems[p])

    def issue(g, p):
        for cp in _group_streams(g, p):
            cp.start()

    def drain(g, p):
        for cp in _group_streams(g, p):
            cp.wait()

    def process(g, carry):
        acc, cnt = carry
        for j in range(GT):
            t = g * GT + j
            acc = tuple(acc[m] + vals_v[t, pl.ds(m * 16, 16)]
                        for m in range(NL))
            cnt = tuple(
                cnt[m]
                + jnp.where(idx_v[t, pl.ds(m * 16, 16)] != 0, 1.0, 0.0)
                .astype(jnp.float32)
                for m in range(NL))
        return acc, cnt

    issue(0, 0)
    issue(1, 1)

    def pair_body(i, carry):
        g0 = i * 2
        for p in range(2):
            g = g0 + p
            drain(g, p)

            @pl.when(g + 2 < NG)
            def _():
                issue(g + 2, p)

            carry = process(g, carry)
        return carry

    zeros = tuple(zerov for _ in range(NL))
    acc, cnt = lax.fori_loop(0, NG // 2, pair_body, (zeros, zeros))

    for m in range(NL):
        countv = cnt[m]
        n0v = seqv - countv
        invv = 1.0 / jnp.maximum(countv, onev)
        lv = (acc[m] - n0v * tw0) * invv + bvec
        e = jnp.exp(lv - lv)
        out_v[pl.ds(m * 16, 16)] = e / e

    pltpu.sync_copy(out_v, out_hbm.at[pl.ds(base, DPW)])


def kernel(documents, table, W, b):
    tw = _table_matvec(table.T, W)
    out = _pool_kernel(documents.astype(jnp.int32).T, tw,
                       jnp.full((16,), b[0], jnp.float32))
    return out.reshape(BATCH, 1)
